# Initial kernel scaffold; baseline (speedup 1.0000x reference)
import jax
import jax.numpy as jnp
from jax.experimental import pallas as pl


def kernel(x, edge_index, batch, mapping, edge_weight, pooling_mask, W1_0, b1_0, g_0, be_0, W2_0, b2_0, eps_0, W1_1, b1_1, g_1, be_1, W2_1, b2_1, eps_1, W1_2, b1_2, g_2, be_2, W2_2, b2_2, eps_2, lin_W, lin_b):
    raise NotImplementedError("write your pallas kernel here")


# speedup vs baseline: 2.6650x; 2.6650x over previous
import functools

import jax
import jax.numpy as jnp
from jax import lax
from jax.experimental import pallas as pl
from jax.experimental.pallas import tpu as pltpu
from jax.experimental.pallas import tpu_sc as plsc

N = 10000
E = 320000
D = 128
H = 128
S = 64
G = 128

NC = 2
NS = 16
NW = NC * NS
EPW = 10240
EP = NW * EPW
K = 80
CH = EPW // K
NP = 10240
RPT = NP // NS
ZR = 32


def _sc_aggregate(h, src1, dst1, ew1):
    mesh = plsc.VectorSubcoreMesh(core_axis_name="c", subcore_axis_name="s",
                                  num_cores=NC, num_subcores=NS)

    @functools.partial(
        pl.kernel,
        out_type=jax.ShapeDtypeStruct((NC, NP, H), jnp.float32),
        mesh=mesh,
        scratch_types=[
            pltpu.VMEM((EPW,), jnp.int32),
            pltpu.VMEM((EPW,), jnp.int32),
            pltpu.VMEM((EPW,), jnp.float32),
            pltpu.VMEM((K,), jnp.int32),
            pltpu.VMEM((K, H), jnp.float32),
            pltpu.VMEM((ZR, H), jnp.float32),
            pltpu.VMEM_SHARED((NP, H), jnp.float32),
        ],
    )
    def agg(h_hbm, src_hbm, dst_hbm, ew_hbm, out_hbm,
            srcv, dstv, ewv, dstc, buf, zbuf, acc):
        cid = lax.axis_index("c")
        sid = lax.axis_index("s")
        wid = sid * NC + cid
        base = wid * EPW

        pltpu.sync_copy(src_hbm.at[pl.ds(base, EPW)], srcv)
        pltpu.sync_copy(dst_hbm.at[pl.ds(base, EPW)], dstv)
        pltpu.sync_copy(ew_hbm.at[pl.ds(base, EPW)], ewv)

        zero = jnp.zeros((16,), jnp.float32)

        def zrow(i, _):
            for c in range(H // 16):
                zbuf[i, pl.ds(c * 16, 16)] = zero
            return 0

        lax.fori_loop(0, ZR, zrow, 0)
        for k in range(RPT // ZR):
            pltpu.sync_copy(zbuf, acc.at[pl.ds(sid * RPT + k * ZR, ZR)])
        plsc.subcore_barrier()

        def chunk(j, _):
            eb = j * K
            pltpu.sync_copy(h_hbm.at[srcv.at[pl.ds(eb, K)]], buf)

            def group(gi, _):
                o = eb + gi * 16
                ewg = ewv[pl.ds(o, 16)]
                dstc[pl.ds(gi * 16, 16)] = dstv[pl.ds(o, 16)]
                for r in range(16):
                    bw = jnp.broadcast_to(ewg[r], (16,))
                    e = gi * 16 + r
                    for c in range(H // 16):
                        sl = pl.ds(c * 16, 16)
                        buf[e, sl] = buf[e, sl] * bw
                return 0

            lax.fori_loop(0, K // 16, group, 0)

            pltpu.sync_copy(buf, acc.at[dstc], add=True)
            return 0

        lax.fori_loop(0, CH, chunk, 0)
        plsc.subcore_barrier()

        for k in range(RPT // ZR):
            s = sid * RPT + k * ZR
            pltpu.sync_copy(acc.at[pl.ds(s, ZR)],
                            out_hbm.at[cid, pl.ds(s, ZR)])

    return agg(h, src1, dst1, ew1)


def _layer_body(h_ref, a_ref, w1_ref, b1_ref, g_ref, be_ref, w2_ref, b2_ref,
                eps_ref, out_ref):
    h = h_ref[...]
    pre = (1.0 + eps_ref[0, 0]) * h + a_ref[0, :N, :] + a_ref[1, :N, :]
    z = jnp.dot(pre, w1_ref[...], preferred_element_type=jnp.float32)
    z = z + b1_ref[...]
    mean = jnp.mean(z, axis=0, keepdims=True)
    var = jnp.mean((z - mean) ** 2, axis=0, keepdims=True)
    z = (z - mean) * lax.rsqrt(var + 1e-5) * g_ref[...] + be_ref[...]
    z = jnp.maximum(z, 0.0)
    z = jnp.dot(z, w2_ref[...], preferred_element_type=jnp.float32)
    z = jnp.maximum(z + b2_ref[...], 0.0)
    out_ref[...] = z


def _tc_layer(h, aggr2, W1, b1, g, be, W2, b2, eps):
    return pl.pallas_call(
        _layer_body,
        out_shape=jax.ShapeDtypeStruct((N, H), jnp.float32),
    )(h, aggr2, W1, b1.reshape(1, H), g.reshape(1, H), be.reshape(1, H),
      W2, b2.reshape(1, H), eps.reshape(1, 1))


def _pool_body(h1_ref, h2_ref, h3_ref, batch_ref, map_ref, mask_ref,
               lw_ref, lb_ref, out_ref):
    hcat = jnp.concatenate([h1_ref[...], h2_ref[...], h3_ref[...]], axis=1)
    iota_gn = lax.broadcasted_iota(jnp.int32, (G, N), 0)
    oh_add = (batch_ref[...] == iota_gn).astype(jnp.float32) * mask_ref[...]
    iota_n = lax.broadcasted_iota(jnp.int32, (G, N), 1)
    oh_ctr = (map_ref[...].reshape(G, 1) == iota_n).astype(jnp.float32)
    ea = jnp.dot(oh_add, hcat, preferred_element_type=jnp.float32)
    ec = jnp.dot(oh_ctr, hcat, preferred_element_type=jnp.float32)
    hp = jnp.concatenate(
        [ea[:, 0:H], ec[:, 0:H], ea[:, H:2 * H], ec[:, H:2 * H],
         ea[:, 2 * H:3 * H], ec[:, 2 * H:3 * H]], axis=1)
    out_ref[...] = (
        jnp.dot(hp, lw_ref[...], preferred_element_type=jnp.float32)
        + lb_ref[...])


def _tc_pool(h1, h2, h3, batch, mapping, mask, lin_W, lin_b):
    return pl.pallas_call(
        _pool_body,
        out_shape=jax.ShapeDtypeStruct((G, S), jnp.float32),
    )(h1, h2, h3, batch.reshape(1, N), mapping.reshape(1, G),
      mask.reshape(1, N), lin_W, lin_b.reshape(1, S))


def kernel(x, edge_index, batch, mapping, edge_weight, pooling_mask,
           W1_0, b1_0, g_0, be_0, W2_0, b2_0, eps_0,
           W1_1, b1_1, g_1, be_1, W2_1, b2_1, eps_1,
           W1_2, b1_2, g_2, be_2, W2_2, b2_2, eps_2,
           lin_W, lin_b):
    pad = EP - E
    zi = jnp.zeros((pad,), jnp.int32)
    src1 = jnp.concatenate([edge_index[0], zi])
    dst1 = jnp.concatenate([edge_index[1], zi])
    ew1 = jnp.concatenate([edge_weight, jnp.zeros((pad,), jnp.float32)])

    layer_params = [
        (W1_0, b1_0, g_0, be_0, W2_0, b2_0, eps_0),
        (W1_1, b1_1, g_1, be_1, W2_1, b2_1, eps_1),
        (W1_2, b1_2, g_2, be_2, W2_2, b2_2, eps_2),
    ]
    h = x
    xs = []
    for (W1, b1, g, be, W2, b2, eps) in layer_params:
        aggr2 = _sc_aggregate(h, src1, dst1, ew1)
        h = _tc_layer(h, aggr2, W1, b1, g, be, W2, b2, eps)
        xs.append(h)
    return _tc_pool(xs[0], xs[1], xs[2], batch, mapping, pooling_mask,
                    lin_W, lin_b)


# speedup vs baseline: 3.1456x; 1.1803x over previous
import functools

import jax
import jax.numpy as jnp
from jax import lax
from jax.experimental import pallas as pl
from jax.experimental.pallas import tpu as pltpu
from jax.experimental.pallas import tpu_sc as plsc

N = 10000
E = 320000
D = 128
H = 128
S = 64
G = 128

NC = 2
NS = 16
NW = NC * NS
EPW = 10240
EP = NW * EPW
K = 80
CH = EPW // K
NP = 10240
RPT = NP // NS
ZR = 32


def _sc_aggregate(h, src1, dst1, ew1):
    mesh = plsc.VectorSubcoreMesh(core_axis_name="c", subcore_axis_name="s",
                                  num_cores=NC, num_subcores=NS)

    @functools.partial(
        pl.kernel,
        out_type=jax.ShapeDtypeStruct((NC, NP, H), jnp.float32),
        mesh=mesh,
        scratch_types=[
            pltpu.VMEM((EPW,), jnp.int32),
            pltpu.VMEM((K,), jnp.int32),
            pltpu.VMEM((K,), jnp.int32),
            pltpu.VMEM((K,), jnp.float32),
            pltpu.VMEM((K,), jnp.float32),
            pltpu.VMEM((K, H), jnp.float32),
            pltpu.VMEM((K, H), jnp.float32),
            pltpu.VMEM((ZR, H), jnp.float32),
            pltpu.VMEM_SHARED((NP, H), jnp.float32),
            pltpu.SemaphoreType.DMA,
            pltpu.SemaphoreType.DMA,
            pltpu.SemaphoreType.DMA,
            pltpu.SemaphoreType.DMA,
            pltpu.SemaphoreType.DMA,
            pltpu.SemaphoreType.DMA,
        ],
    )
    def agg(h_hbm, src_hbm, dst_hbm, ew_hbm, out_hbm,
            srcv, dst0, dst1, ew0, ew1, buf0, buf1, zbuf, acc,
            g0, g1, s0, s1, m0, m1):
        cid = lax.axis_index("c")
        sid = lax.axis_index("s")
        wid = sid * NC + cid
        base = wid * EPW

        pltpu.sync_copy(src_hbm.at[pl.ds(base, EPW)], srcv)

        zero = jnp.zeros((16,), jnp.float32)

        def zrow(i, _):
            for c in range(H // 16):
                zbuf[i, pl.ds(c * 16, 16)] = zero
            return 0

        lax.fori_loop(0, ZR, zrow, 0)
        for k in range(RPT // ZR):
            pltpu.sync_copy(zbuf, acc.at[pl.ds(sid * RPT + k * ZR, ZR)])
        plsc.subcore_barrier()

        bufs = (buf0, buf1)
        dsts = (dst0, dst1)
        ews = (ew0, ew1)
        gsems = (g0, g1)
        ssems = (s0, s1)
        msems = (m0, m1)

        def issue_fetch(j, b):
            eb = j * K
            pltpu.async_copy(h_hbm.at[srcv.at[pl.ds(eb, K)]], bufs[b],
                             gsems[b])
            pltpu.async_copy(dst_hbm.at[pl.ds(base + eb, K)], dsts[b],
                             msems[b])
            pltpu.async_copy(ew_hbm.at[pl.ds(base + eb, K)], ews[b],
                             msems[b])

        def wait_fetch(j, b):
            eb = j * K
            pltpu.make_async_copy(h_hbm.at[srcv.at[pl.ds(eb, K)]], bufs[b],
                                  gsems[b]).wait()
            pltpu.make_async_copy(dst_hbm.at[pl.ds(base + eb, K)], dsts[b],
                                  msems[b]).wait()
            pltpu.make_async_copy(ew_hbm.at[pl.ds(base + eb, K)], ews[b],
                                  msems[b]).wait()

        def scale(b):
            buf = bufs[b]
            ewv = ews[b]

            def group(gi, _):
                ewg = ewv[pl.ds(gi * 16, 16)]
                for r in range(16):
                    bw = jnp.broadcast_to(ewg[r], (16,))
                    e = gi * 16 + r
                    for c in range(H // 16):
                        sl = pl.ds(c * 16, 16)
                        buf[e, sl] = buf[e, sl] * bw
                return 0

            lax.fori_loop(0, K // 16, group, 0)

        issue_fetch(0, 0)
        issue_fetch(1, 1)

        def body(jj, _):
            for b in range(2):
                j = jj * 2 + b
                wait_fetch(j, b)
                scale(b)
                pltpu.async_copy(bufs[b], acc.at[dsts[b]], ssems[b],
                                 add=True)
            for b in range(2):
                j = jj * 2 + b
                pltpu.make_async_copy(bufs[b], acc.at[dsts[b]],
                                      ssems[b]).wait()

                @pl.when(j + 2 < CH)
                def _():
                    issue_fetch(j + 2, b)
            return 0

        lax.fori_loop(0, CH // 2, body, 0)
        plsc.subcore_barrier()

        for k in range(RPT // ZR):
            s = sid * RPT + k * ZR
            pltpu.sync_copy(acc.at[pl.ds(s, ZR)],
                            out_hbm.at[cid, pl.ds(s, ZR)])

    return agg(h, src1, dst1, ew1)


def _layer_body(h_ref, a_ref, w1_ref, b1_ref, g_ref, be_ref, w2_ref, b2_ref,
                eps_ref, out_ref):
    h = h_ref[...]
    pre = (1.0 + eps_ref[0, 0]) * h + a_ref[0, :N, :] + a_ref[1, :N, :]
    z = jnp.dot(pre, w1_ref[...], preferred_element_type=jnp.float32)
    z = z + b1_ref[...]
    mean = jnp.mean(z, axis=0, keepdims=True)
    var = jnp.mean((z - mean) ** 2, axis=0, keepdims=True)
    z = (z - mean) * lax.rsqrt(var + 1e-5) * g_ref[...] + be_ref[...]
    z = jnp.maximum(z, 0.0)
    z = jnp.dot(z, w2_ref[...], preferred_element_type=jnp.float32)
    z = jnp.maximum(z + b2_ref[...], 0.0)
    out_ref[...] = z


def _tc_layer(h, aggr2, W1, b1, g, be, W2, b2, eps):
    return pl.pallas_call(
        _layer_body,
        out_shape=jax.ShapeDtypeStruct((N, H), jnp.float32),
    )(h, aggr2, W1, b1.reshape(1, H), g.reshape(1, H), be.reshape(1, H),
      W2, b2.reshape(1, H), eps.reshape(1, 1))


def _pool_body(h1_ref, h2_ref, h3_ref, batch_ref, map_ref, mask_ref,
               lw_ref, lb_ref, out_ref):
    hcat = jnp.concatenate([h1_ref[...], h2_ref[...], h3_ref[...]], axis=1)
    iota_gn = lax.broadcasted_iota(jnp.int32, (G, N), 0)
    oh_add = (batch_ref[...] == iota_gn).astype(jnp.float32) * mask_ref[...]
    iota_n = lax.broadcasted_iota(jnp.int32, (G, N), 1)
    oh_ctr = (map_ref[...].reshape(G, 1) == iota_n).astype(jnp.float32)
    ea = jnp.dot(oh_add, hcat, preferred_element_type=jnp.float32)
    ec = jnp.dot(oh_ctr, hcat, preferred_element_type=jnp.float32)
    hp = jnp.concatenate(
        [ea[:, 0:H], ec[:, 0:H], ea[:, H:2 * H], ec[:, H:2 * H],
         ea[:, 2 * H:3 * H], ec[:, 2 * H:3 * H]], axis=1)
    out_ref[...] = (
        jnp.dot(hp, lw_ref[...], preferred_element_type=jnp.float32)
        + lb_ref[...])


def _tc_pool(h1, h2, h3, batch, mapping, mask, lin_W, lin_b):
    return pl.pallas_call(
        _pool_body,
        out_shape=jax.ShapeDtypeStruct((G, S), jnp.float32),
    )(h1, h2, h3, batch.reshape(1, N), mapping.reshape(1, G),
      mask.reshape(1, N), lin_W, lin_b.reshape(1, S))


def kernel(x, edge_index, batch, mapping, edge_weight, pooling_mask,
           W1_0, b1_0, g_0, be_0, W2_0, b2_0, eps_0,
           W1_1, b1_1, g_1, be_1, W2_1, b2_1, eps_1,
           W1_2, b1_2, g_2, be_2, W2_2, b2_2, eps_2,
           lin_W, lin_b):
    pad = EP - E
    zi = jnp.zeros((pad,), jnp.int32)
    src1 = jnp.concatenate([edge_index[0], zi])
    dst1 = jnp.concatenate([edge_index[1], zi])
    ew1 = jnp.concatenate([edge_weight, jnp.zeros((pad,), jnp.float32)])

    layer_params = [
        (W1_0, b1_0, g_0, be_0, W2_0, b2_0, eps_0),
        (W1_1, b1_1, g_1, be_1, W2_1, b2_1, eps_1),
        (W1_2, b1_2, g_2, be_2, W2_2, b2_2, eps_2),
    ]
    h = x
    xs = []
    for (W1, b1, g, be, W2, b2, eps) in layer_params:
        aggr2 = _sc_aggregate(h, src1, dst1, ew1)
        h = _tc_layer(h, aggr2, W1, b1, g, be, W2, b2, eps)
        xs.append(h)
    return _tc_pool(xs[0], xs[1], xs[2], batch, mapping, pooling_mask,
                    lin_W, lin_b)
